# Initial kernel scaffold; baseline (speedup 1.0000x reference)
import jax
import jax.numpy as jnp
from jax.experimental import pallas as pl


def kernel(tokens_a, len_a, tokens_b, len_b):
    raise NotImplementedError("write your pallas kernel here")


# speedup vs baseline: 1.3618x; 1.3618x over previous
import functools

import jax
import jax.numpy as jnp
from jax import lax
from jax.experimental import pallas as pl
from jax.experimental.pallas import tpu as pltpu
from jax.experimental.pallas import tpu_sc as plsc

SEQ_LEN = 512
BATCH = 16
LANES = 16
HALF = SEQ_LEN // 2
START_VALUE = 101
END_VALUE = 102
BUDGET = SEQ_LEN - 3


def _packer_body(tokens_a_hbm, len_a_hbm, tokens_b_hbm, len_b_hbm,
                 out_tok_hbm, out_mask_hbm, out_seg_hbm,
                 a_v, b_v, la_v, lb_v, tok_v, mask_v, seg_v):
    core = lax.axis_index("c")
    row = lax.axis_index("s")

    pltpu.sync_copy(tokens_a_hbm.at[row], a_v)
    pltpu.sync_copy(tokens_b_hbm.at[row], b_v)
    pltpu.sync_copy(len_a_hbm, la_v)
    pltpu.sync_copy(len_b_hbm, lb_v)

    row_idx = jnp.zeros((LANES,), jnp.int32) + row
    la = plsc.load_gather(la_v, [row_idx])
    lb = plsc.load_gather(lb_v, [row_idx])

    l1 = jnp.minimum(la, BUDGET)
    l2 = jnp.minimum(lb, BUDGET - l1)

    base = core * HALF
    for ci in range(HALF // LANES):
        pos = base + ci * LANES + lax.iota(jnp.int32, LANES)
        idx_a = jnp.clip(pos - 1, 0, SEQ_LEN - 1)
        ga = plsc.load_gather(a_v, [idx_a])
        idx_b = jnp.clip(pos - (l1 + 2), 0, SEQ_LEN - 1)
        gb = plsc.load_gather(b_v, [idx_b])

        is_start = pos == 0
        is_a = (pos >= 1) & (pos <= l1)
        is_end1 = pos == l1 + 1
        is_b = (pos >= l1 + 2) & (pos <= l1 + 1 + l2)
        is_end2 = pos == l1 + l2 + 2

        tok = jnp.where(
            is_start, START_VALUE,
            jnp.where(is_a, ga,
                      jnp.where(is_end1, END_VALUE,
                                jnp.where(is_b, gb,
                                          jnp.where(is_end2, END_VALUE, 0)))))
        valid = is_start | is_a | is_end1 | is_b | is_end2
        seg = is_b | is_end2

        sl = pl.ds(ci * LANES, LANES)
        tok_v[sl] = tok.astype(jnp.int32)
        mask_v[sl] = valid.astype(jnp.int32)
        seg_v[sl] = seg.astype(jnp.int32)

    pltpu.sync_copy(tok_v, out_tok_hbm.at[row, pl.ds(base, HALF)])
    pltpu.sync_copy(mask_v, out_mask_hbm.at[row, pl.ds(base, HALF)])
    pltpu.sync_copy(seg_v, out_seg_hbm.at[row, pl.ds(base, HALF)])


_packer = functools.partial(
    pl.kernel,
    out_type=(
        jax.ShapeDtypeStruct((BATCH, SEQ_LEN), jnp.int32),
        jax.ShapeDtypeStruct((BATCH, SEQ_LEN), jnp.int32),
        jax.ShapeDtypeStruct((BATCH, SEQ_LEN), jnp.int32),
    ),
    mesh=plsc.VectorSubcoreMesh(
        core_axis_name="c", subcore_axis_name="s",
        num_cores=2, num_subcores=16),
    scratch_types=[
        pltpu.VMEM((SEQ_LEN,), jnp.int32),
        pltpu.VMEM((SEQ_LEN,), jnp.int32),
        pltpu.VMEM((BATCH,), jnp.int32),
        pltpu.VMEM((BATCH,), jnp.int32),
        pltpu.VMEM((HALF,), jnp.int32),
        pltpu.VMEM((HALF,), jnp.int32),
        pltpu.VMEM((HALF,), jnp.int32),
    ],
    compiler_params=pltpu.CompilerParams(needs_layout_passes=False),
)(_packer_body)


def kernel(tokens_a, len_a, tokens_b, len_b):
    return _packer(tokens_a, len_a, tokens_b, len_b)


# speedup vs baseline: 1.4766x; 1.0843x over previous
import functools

import jax
import jax.numpy as jnp
from jax import lax
from jax.experimental import pallas as pl
from jax.experimental.pallas import tpu as pltpu
from jax.experimental.pallas import tpu_sc as plsc

SEQ_LEN = 512
BATCH = 16
LANES = 16
HALF = SEQ_LEN // 2
START_VALUE = 101
END_VALUE = 102
BUDGET = SEQ_LEN - 3


def _packer_body(tokens_a_hbm, len_a_hbm, tokens_b_hbm, len_b_hbm,
                 out_tok_hbm, out_mask_hbm, out_seg_hbm,
                 a_v, b_v, la_v, lb_v, tok_v, mask_v, seg_v, sem):
    core = lax.axis_index("c")
    row = lax.axis_index("s")

    cp_la = pltpu.async_copy(len_a_hbm, la_v, sem)
    cp_lb = pltpu.async_copy(len_b_hbm, lb_v, sem)
    cp_a = pltpu.async_copy(tokens_a_hbm.at[row], a_v, sem)
    cp_b = pltpu.async_copy(tokens_b_hbm.at[row], b_v, sem)
    cp_la.wait()
    cp_lb.wait()

    zeros = jnp.zeros((LANES,), jnp.int32)
    row_idx = zeros + row
    la = plsc.load_gather(la_v, [row_idx])
    lb = plsc.load_gather(lb_v, [row_idx])
    l1 = jnp.minimum(la, BUDGET)
    l2 = jnp.minimum(lb, BUDGET - l1)

    base = core * HALF

    cp_a.wait()
    cp_b.wait()

    for ci in range(HALF // LANES):
        pos = base + ci * LANES + lax.iota(jnp.int32, LANES)
        idx_a = jnp.clip(pos - 1, 0, SEQ_LEN - 1)
        ga = plsc.load_gather(a_v, [idx_a])
        idx_b = jnp.clip(pos - (l1 + 2), 0, SEQ_LEN - 1)
        gb = plsc.load_gather(b_v, [idx_b])

        is_start = pos == 0
        is_a = (pos >= 1) & (pos <= l1)
        is_end1 = pos == l1 + 1
        is_b = (pos >= l1 + 2) & (pos <= l1 + 1 + l2)
        is_end2 = pos == l1 + l2 + 2

        tok = jnp.where(
            is_start, START_VALUE,
            jnp.where(is_a, ga,
                      jnp.where(is_end1, END_VALUE,
                                jnp.where(is_b, gb,
                                          jnp.where(is_end2, END_VALUE, 0)))))
        valid = is_start | is_a | is_end1 | is_b | is_end2
        seg = is_b | is_end2

        sl = pl.ds(ci * LANES, LANES)
        tok_v[sl] = tok.astype(jnp.int32)
        mask_v[sl] = valid.astype(jnp.int32)
        seg_v[sl] = seg.astype(jnp.int32)

    cp_t = pltpu.async_copy(tok_v, out_tok_hbm.at[row, pl.ds(base, HALF)], sem)
    cp_m = pltpu.async_copy(mask_v, out_mask_hbm.at[row, pl.ds(base, HALF)], sem)
    cp_s = pltpu.async_copy(seg_v, out_seg_hbm.at[row, pl.ds(base, HALF)], sem)
    cp_t.wait()
    cp_m.wait()
    cp_s.wait()


_packer = functools.partial(
    pl.kernel,
    out_type=(
        jax.ShapeDtypeStruct((BATCH, SEQ_LEN), jnp.int32),
        jax.ShapeDtypeStruct((BATCH, SEQ_LEN), jnp.int32),
        jax.ShapeDtypeStruct((BATCH, SEQ_LEN), jnp.int32),
    ),
    mesh=plsc.VectorSubcoreMesh(
        core_axis_name="c", subcore_axis_name="s",
        num_cores=2, num_subcores=16),
    scratch_types=[
        pltpu.VMEM((SEQ_LEN,), jnp.int32),
        pltpu.VMEM((SEQ_LEN,), jnp.int32),
        pltpu.VMEM((BATCH,), jnp.int32),
        pltpu.VMEM((BATCH,), jnp.int32),
        pltpu.VMEM((HALF,), jnp.int32),
        pltpu.VMEM((HALF,), jnp.int32),
        pltpu.VMEM((HALF,), jnp.int32),
        pltpu.SemaphoreType.DMA,
    ],
    compiler_params=pltpu.CompilerParams(needs_layout_passes=False),
)(_packer_body)


def kernel(tokens_a, len_a, tokens_b, len_b):
    return _packer(tokens_a, len_a, tokens_b, len_b)


# speedup vs baseline: 1.4794x; 1.0019x over previous
import functools

import jax
import jax.numpy as jnp
from jax import lax
from jax.experimental import pallas as pl
from jax.experimental.pallas import tpu as pltpu
from jax.experimental.pallas import tpu_sc as plsc

SEQ_LEN = 512
BATCH = 16
LANES = 16
HALF = SEQ_LEN // 2
START_VALUE = 101
END_VALUE = 102
BUDGET = SEQ_LEN - 3


def _packer_body(tokens_a_hbm, len_a_hbm, tokens_b_hbm, len_b_hbm,
                 out_tok_hbm, out_mask_hbm, out_seg_hbm,
                 a_v, b_v, la_v, lb_v, tok_v, mask_v, seg_v, sem):
    core = lax.axis_index("c")
    row = lax.axis_index("s")

    cp_la = pltpu.async_copy(len_a_hbm, la_v, sem)
    cp_lb = pltpu.async_copy(len_b_hbm, lb_v, sem)
    cp_a = pltpu.async_copy(tokens_a_hbm.at[row], a_v, sem)
    cp_b = pltpu.async_copy(tokens_b_hbm.at[row], b_v, sem)
    cp_la.wait()
    cp_lb.wait()

    zeros = jnp.zeros((LANES,), jnp.int32)
    row_idx = zeros + row
    la = plsc.load_gather(la_v, [row_idx])
    lb = plsc.load_gather(lb_v, [row_idx])
    l1 = jnp.minimum(la, BUDGET)
    l2 = jnp.minimum(lb, BUDGET - l1)

    base = core * HALF

    cp_a.wait()
    cp_b.wait()

    def chunk(ci, _):
        pos = base + ci * LANES + lax.iota(jnp.int32, LANES)
        idx_a = jnp.clip(pos - 1, 0, SEQ_LEN - 1)
        ga = plsc.load_gather(a_v, [idx_a])
        idx_b = jnp.clip(pos - (l1 + 2), 0, SEQ_LEN - 1)
        gb = plsc.load_gather(b_v, [idx_b])

        is_start = pos == 0
        is_a = (pos >= 1) & (pos <= l1)
        is_end1 = pos == l1 + 1
        is_b = (pos >= l1 + 2) & (pos <= l1 + 1 + l2)
        is_end2 = pos == l1 + l2 + 2

        tok = jnp.where(
            is_start, START_VALUE,
            jnp.where(is_a, ga,
                      jnp.where(is_end1, END_VALUE,
                                jnp.where(is_b, gb,
                                          jnp.where(is_end2, END_VALUE, 0)))))
        valid = is_start | is_a | is_end1 | is_b | is_end2
        seg = is_b | is_end2

        sl = pl.ds(ci * LANES, LANES)
        tok_v[sl] = tok.astype(jnp.int32)
        mask_v[sl] = valid.astype(jnp.int32)
        seg_v[sl] = seg.astype(jnp.int32)
        return _

    lax.fori_loop(0, HALF // LANES, chunk, None, unroll=4)

    cp_t = pltpu.async_copy(tok_v, out_tok_hbm.at[row, pl.ds(base, HALF)], sem)
    cp_m = pltpu.async_copy(mask_v, out_mask_hbm.at[row, pl.ds(base, HALF)], sem)
    cp_s = pltpu.async_copy(seg_v, out_seg_hbm.at[row, pl.ds(base, HALF)], sem)
    cp_t.wait()
    cp_m.wait()
    cp_s.wait()


_packer = functools.partial(
    pl.kernel,
    out_type=(
        jax.ShapeDtypeStruct((BATCH, SEQ_LEN), jnp.int32),
        jax.ShapeDtypeStruct((BATCH, SEQ_LEN), jnp.int32),
        jax.ShapeDtypeStruct((BATCH, SEQ_LEN), jnp.int32),
    ),
    mesh=plsc.VectorSubcoreMesh(
        core_axis_name="c", subcore_axis_name="s",
        num_cores=2, num_subcores=16),
    scratch_types=[
        pltpu.VMEM((SEQ_LEN,), jnp.int32),
        pltpu.VMEM((SEQ_LEN,), jnp.int32),
        pltpu.VMEM((BATCH,), jnp.int32),
        pltpu.VMEM((BATCH,), jnp.int32),
        pltpu.VMEM((HALF,), jnp.int32),
        pltpu.VMEM((HALF,), jnp.int32),
        pltpu.VMEM((HALF,), jnp.int32),
        pltpu.SemaphoreType.DMA,
    ],
    compiler_params=pltpu.CompilerParams(needs_layout_passes=False),
)(_packer_body)


def kernel(tokens_a, len_a, tokens_b, len_b):
    return _packer(tokens_a, len_a, tokens_b, len_b)


# speedup vs baseline: 1.4909x; 1.0077x over previous
import functools

import jax
import jax.numpy as jnp
from jax import lax
from jax.experimental import pallas as pl
from jax.experimental.pallas import tpu as pltpu
from jax.experimental.pallas import tpu_sc as plsc

SEQ_LEN = 512
BATCH = 16
LANES = 16
HALF = SEQ_LEN // 2
START_VALUE = 101
END_VALUE = 102
BUDGET = SEQ_LEN - 3


def _packer_body(tokens_a_hbm, len_a_hbm, tokens_b_hbm, len_b_hbm,
                 out_tok_hbm, out_mask_hbm, out_seg_hbm,
                 a_v, b_v, la_v, lb_v, tok_v, mask_v, seg_v, sem):
    core = lax.axis_index("c")
    row = lax.axis_index("s")

    cp_la = pltpu.async_copy(len_a_hbm, la_v, sem)
    cp_lb = pltpu.async_copy(len_b_hbm, lb_v, sem)
    cp_a = pltpu.async_copy(tokens_a_hbm.at[row], a_v, sem)
    cp_b = pltpu.async_copy(tokens_b_hbm.at[row], b_v, sem)
    cp_la.wait()
    cp_lb.wait()

    zeros = jnp.zeros((LANES,), jnp.int32)
    row_idx = zeros + row
    la = plsc.load_gather(la_v, [row_idx])
    lb = plsc.load_gather(lb_v, [row_idx])
    l1 = jnp.minimum(la, BUDGET)
    l2 = jnp.minimum(lb, BUDGET - l1)
    b0 = l1 + 2
    e = b0 + l2
    zero = jnp.int32(0)

    base = core * HALF

    cp_a.wait()
    cp_b.wait()

    def chunk(ci, _):
        pos = base + ci * LANES + lax.iota(jnp.int32, LANES)
        ga = plsc.load_gather(a_v, [jnp.maximum(pos - 1, zero)])
        gb = plsc.load_gather(b_v, [jnp.maximum(pos - b0, zero)])

        tok = jnp.where(
            pos == 0, START_VALUE,
            jnp.where(pos <= l1, ga,
                      jnp.where(pos == l1 + 1, END_VALUE,
                                jnp.where(pos < e, gb,
                                          jnp.where(pos == e, END_VALUE, 0)))))

        sl = pl.ds(ci * LANES, LANES)
        tok_v[sl] = tok.astype(jnp.int32)
        mask_v[sl] = (pos <= e).astype(jnp.int32)
        seg_v[sl] = ((pos >= b0) & (pos <= e)).astype(jnp.int32)
        return _

    lax.fori_loop(0, HALF // LANES, chunk, None, unroll=4)

    cp_t = pltpu.async_copy(tok_v, out_tok_hbm.at[row, pl.ds(base, HALF)], sem)
    cp_m = pltpu.async_copy(mask_v, out_mask_hbm.at[row, pl.ds(base, HALF)], sem)
    cp_s = pltpu.async_copy(seg_v, out_seg_hbm.at[row, pl.ds(base, HALF)], sem)
    cp_t.wait()
    cp_m.wait()
    cp_s.wait()


_packer = functools.partial(
    pl.kernel,
    out_type=(
        jax.ShapeDtypeStruct((BATCH, SEQ_LEN), jnp.int32),
        jax.ShapeDtypeStruct((BATCH, SEQ_LEN), jnp.int32),
        jax.ShapeDtypeStruct((BATCH, SEQ_LEN), jnp.int32),
    ),
    mesh=plsc.VectorSubcoreMesh(
        core_axis_name="c", subcore_axis_name="s",
        num_cores=2, num_subcores=16),
    scratch_types=[
        pltpu.VMEM((SEQ_LEN,), jnp.int32),
        pltpu.VMEM((SEQ_LEN,), jnp.int32),
        pltpu.VMEM((BATCH,), jnp.int32),
        pltpu.VMEM((BATCH,), jnp.int32),
        pltpu.VMEM((HALF,), jnp.int32),
        pltpu.VMEM((HALF,), jnp.int32),
        pltpu.VMEM((HALF,), jnp.int32),
        pltpu.SemaphoreType.DMA,
    ],
    compiler_params=pltpu.CompilerParams(needs_layout_passes=False),
)(_packer_body)


def kernel(tokens_a, len_a, tokens_b, len_b):
    return _packer(tokens_a, len_a, tokens_b, len_b)


# speedup vs baseline: 1.5024x; 1.0077x over previous
import functools

import jax
import jax.numpy as jnp
from jax import lax
from jax.experimental import pallas as pl
from jax.experimental.pallas import tpu as pltpu
from jax.experimental.pallas import tpu_sc as plsc

SEQ_LEN = 512
BATCH = 16
LANES = 16
HALF = SEQ_LEN // 2
START_VALUE = 101
END_VALUE = 102
BUDGET = SEQ_LEN - 3


def _packer_body(tokens_a_hbm, len_a_hbm, tokens_b_hbm, len_b_hbm,
                 out_tok_hbm, out_mask_hbm, out_seg_hbm,
                 a_v, b_v, la_v, lb_v, tok_v, mask_v, seg_v, sem):
    core = lax.axis_index("c")
    row = lax.axis_index("s")

    cp_la = pltpu.async_copy(len_a_hbm, la_v, sem)
    cp_lb = pltpu.async_copy(len_b_hbm, lb_v, sem)
    cp_a = pltpu.async_copy(tokens_a_hbm.at[row], a_v, sem)
    cp_b = pltpu.async_copy(tokens_b_hbm.at[row], b_v, sem)
    cp_la.wait()
    cp_lb.wait()

    zeros = jnp.zeros((LANES,), jnp.int32)
    row_idx = zeros + row
    la = plsc.load_gather(la_v, [row_idx])
    lb = plsc.load_gather(lb_v, [row_idx])
    l1 = jnp.minimum(la, BUDGET)
    l2 = jnp.minimum(lb, BUDGET - l1)
    b0 = l1 + 2
    e = b0 + l2
    zero = jnp.int32(0)

    base = core * HALF

    def mask_chunk(ci, _):
        pos = base + ci * LANES + lax.iota(jnp.int32, LANES)
        sl = pl.ds(ci * LANES, LANES)
        mask_v[sl] = (pos <= e).astype(jnp.int32)
        seg_v[sl] = ((pos >= b0) & (pos <= e)).astype(jnp.int32)
        return _

    lax.fori_loop(0, HALF // LANES, mask_chunk, None, unroll=4)
    cp_m = pltpu.async_copy(mask_v, out_mask_hbm.at[row, pl.ds(base, HALF)], sem)
    cp_s = pltpu.async_copy(seg_v, out_seg_hbm.at[row, pl.ds(base, HALF)], sem)

    cp_a.wait()
    cp_b.wait()

    def tok_chunk(ci, _):
        pos = base + ci * LANES + lax.iota(jnp.int32, LANES)
        ga = plsc.load_gather(a_v, [jnp.maximum(pos - 1, zero)])
        gb = plsc.load_gather(b_v, [jnp.maximum(pos - b0, zero)])

        tok = jnp.where(
            pos == 0, START_VALUE,
            jnp.where(pos <= l1, ga,
                      jnp.where(pos == l1 + 1, END_VALUE,
                                jnp.where(pos < e, gb,
                                          jnp.where(pos == e, END_VALUE, 0)))))

        tok_v[pl.ds(ci * LANES, LANES)] = tok.astype(jnp.int32)
        return _

    lax.fori_loop(0, HALF // LANES, tok_chunk, None, unroll=4)

    cp_t = pltpu.async_copy(tok_v, out_tok_hbm.at[row, pl.ds(base, HALF)], sem)
    cp_m.wait()
    cp_s.wait()
    cp_t.wait()


_packer = functools.partial(
    pl.kernel,
    out_type=(
        jax.ShapeDtypeStruct((BATCH, SEQ_LEN), jnp.int32),
        jax.ShapeDtypeStruct((BATCH, SEQ_LEN), jnp.int32),
        jax.ShapeDtypeStruct((BATCH, SEQ_LEN), jnp.int32),
    ),
    mesh=plsc.VectorSubcoreMesh(
        core_axis_name="c", subcore_axis_name="s",
        num_cores=2, num_subcores=16),
    scratch_types=[
        pltpu.VMEM((SEQ_LEN,), jnp.int32),
        pltpu.VMEM((SEQ_LEN,), jnp.int32),
        pltpu.VMEM((BATCH,), jnp.int32),
        pltpu.VMEM((BATCH,), jnp.int32),
        pltpu.VMEM((HALF,), jnp.int32),
        pltpu.VMEM((HALF,), jnp.int32),
        pltpu.VMEM((HALF,), jnp.int32),
        pltpu.SemaphoreType.DMA,
    ],
    compiler_params=pltpu.CompilerParams(needs_layout_passes=False),
)(_packer_body)


def kernel(tokens_a, len_a, tokens_b, len_b):
    return _packer(tokens_a, len_a, tokens_b, len_b)


# speedup vs baseline: 1.5102x; 1.0052x over previous
import functools

import jax
import jax.numpy as jnp
from jax import lax
from jax.experimental import pallas as pl
from jax.experimental.pallas import tpu as pltpu
from jax.experimental.pallas import tpu_sc as plsc

SEQ_LEN = 512
BATCH = 16
LANES = 16
HALF = SEQ_LEN // 2
START_VALUE = 101
END_VALUE = 102
BUDGET = SEQ_LEN - 3


def _packer_body(tokens_a_hbm, len_a_hbm, tokens_b_hbm, len_b_hbm,
                 out_tok_hbm, out_mask_hbm, out_seg_hbm,
                 a_v, b_v, la_v, lb_v, tok_v, mask_v, seg_v, sem):
    core = lax.axis_index("c")
    row = lax.axis_index("s")

    cp_la = pltpu.async_copy(len_a_hbm, la_v, sem)
    cp_lb = pltpu.async_copy(len_b_hbm, lb_v, sem)
    cp_a = pltpu.async_copy(tokens_a_hbm.at[row], a_v, sem)
    cp_b = pltpu.async_copy(tokens_b_hbm.at[row], b_v, sem)
    cp_la.wait()
    cp_lb.wait()

    zeros = jnp.zeros((LANES,), jnp.int32)
    row_idx = zeros + row
    la = plsc.load_gather(la_v, [row_idx])
    lb = plsc.load_gather(lb_v, [row_idx])
    l1 = jnp.minimum(la, BUDGET)
    l2 = jnp.minimum(lb, BUDGET - l1)
    b0 = l1 + 2
    e = b0 + l2
    zero = jnp.int32(0)

    base = core * HALF

    def mask_chunk(ci, _):
        pos = base + ci * LANES + lax.iota(jnp.int32, LANES)
        sl = pl.ds(ci * LANES, LANES)
        mask_v[sl] = (pos <= e).astype(jnp.int32)
        seg_v[sl] = ((pos >= b0) & (pos <= e)).astype(jnp.int32)
        return _

    lax.fori_loop(0, HALF // LANES, mask_chunk, None, unroll=1)
    cp_m = pltpu.async_copy(mask_v, out_mask_hbm.at[row, pl.ds(base, HALF)], sem)
    cp_s = pltpu.async_copy(seg_v, out_seg_hbm.at[row, pl.ds(base, HALF)], sem)

    cp_a.wait()
    cp_b.wait()

    def tok_chunk(ci, _):
        pos = base + ci * LANES + lax.iota(jnp.int32, LANES)
        ga = plsc.load_gather(a_v, [jnp.maximum(pos - 1, zero)])
        gb = plsc.load_gather(b_v, [jnp.maximum(pos - b0, zero)])

        tok = jnp.where(
            pos == 0, START_VALUE,
            jnp.where(pos <= l1, ga,
                      jnp.where(pos == l1 + 1, END_VALUE,
                                jnp.where(pos < e, gb,
                                          jnp.where(pos == e, END_VALUE, 0)))))

        tok_v[pl.ds(ci * LANES, LANES)] = tok.astype(jnp.int32)
        return _

    lax.fori_loop(0, HALF // LANES, tok_chunk, None, unroll=1)

    cp_t = pltpu.async_copy(tok_v, out_tok_hbm.at[row, pl.ds(base, HALF)], sem)
    cp_m.wait()
    cp_s.wait()
    cp_t.wait()


_packer = functools.partial(
    pl.kernel,
    out_type=(
        jax.ShapeDtypeStruct((BATCH, SEQ_LEN), jnp.int32),
        jax.ShapeDtypeStruct((BATCH, SEQ_LEN), jnp.int32),
        jax.ShapeDtypeStruct((BATCH, SEQ_LEN), jnp.int32),
    ),
    mesh=plsc.VectorSubcoreMesh(
        core_axis_name="c", subcore_axis_name="s",
        num_cores=2, num_subcores=16),
    scratch_types=[
        pltpu.VMEM((SEQ_LEN,), jnp.int32),
        pltpu.VMEM((SEQ_LEN,), jnp.int32),
        pltpu.VMEM((BATCH,), jnp.int32),
        pltpu.VMEM((BATCH,), jnp.int32),
        pltpu.VMEM((HALF,), jnp.int32),
        pltpu.VMEM((HALF,), jnp.int32),
        pltpu.VMEM((HALF,), jnp.int32),
        pltpu.SemaphoreType.DMA,
    ],
    compiler_params=pltpu.CompilerParams(needs_layout_passes=False),
)(_packer_body)


def kernel(tokens_a, len_a, tokens_b, len_b):
    return _packer(tokens_a, len_a, tokens_b, len_b)


# speedup vs baseline: 1.5149x; 1.0031x over previous
import functools

import jax
import jax.numpy as jnp
from jax import lax
from jax.experimental import pallas as pl
from jax.experimental.pallas import tpu as pltpu
from jax.experimental.pallas import tpu_sc as plsc

SEQ_LEN = 512
BATCH = 16
LANES = 16
HALF = SEQ_LEN // 2
START_VALUE = 101
END_VALUE = 102
BUDGET = SEQ_LEN - 3


def _packer_body(tokens_a_hbm, len_a_hbm, tokens_b_hbm, len_b_hbm,
                 out_tok_hbm, out_mask_hbm, out_seg_hbm,
                 a_v, b_v, la_v, lb_v, tok_v, mask_v, seg_v, sem):
    core = lax.axis_index("c")
    row = lax.axis_index("s")

    cp_la = pltpu.async_copy(len_a_hbm, la_v, sem)
    cp_lb = pltpu.async_copy(len_b_hbm, lb_v, sem)
    cp_a = pltpu.async_copy(tokens_a_hbm.at[row], a_v, sem)
    cp_b = pltpu.async_copy(tokens_b_hbm.at[row], b_v, sem)
    cp_la.wait()
    cp_lb.wait()

    zeros = jnp.zeros((LANES,), jnp.int32)
    row_idx = zeros + row
    la = plsc.load_gather(la_v, [row_idx])
    lb = plsc.load_gather(lb_v, [row_idx])
    l1 = jnp.minimum(la, BUDGET)
    l2 = jnp.minimum(lb, BUDGET - l1)
    b0 = l1 + 2
    e = b0 + l2
    zero = jnp.int32(0)

    base = core * HALF

    @plsc.parallel_loop(0, HALF, LANES, unroll=2)
    def mask_chunk(off):
        pos = base + off + lax.iota(jnp.int32, LANES)
        sl = pl.ds(off, LANES)
        mask_v[sl] = (pos <= e).astype(jnp.int32)
        seg_v[sl] = ((pos >= b0) & (pos <= e)).astype(jnp.int32)
    cp_m = pltpu.async_copy(mask_v, out_mask_hbm.at[row, pl.ds(base, HALF)], sem)
    cp_s = pltpu.async_copy(seg_v, out_seg_hbm.at[row, pl.ds(base, HALF)], sem)

    cp_a.wait()
    cp_b.wait()

    @plsc.parallel_loop(0, HALF, LANES, unroll=2)
    def tok_chunk(off):
        pos = base + off + lax.iota(jnp.int32, LANES)
        ga = plsc.load_gather(a_v, [jnp.maximum(pos - 1, zero)])
        gb = plsc.load_gather(b_v, [jnp.maximum(pos - b0, zero)])

        tok = jnp.where(
            pos == 0, START_VALUE,
            jnp.where(pos <= l1, ga,
                      jnp.where(pos == l1 + 1, END_VALUE,
                                jnp.where(pos < e, gb,
                                          jnp.where(pos == e, END_VALUE, 0)))))

        tok_v[pl.ds(off, LANES)] = tok.astype(jnp.int32)

    cp_t = pltpu.async_copy(tok_v, out_tok_hbm.at[row, pl.ds(base, HALF)], sem)
    cp_m.wait()
    cp_s.wait()
    cp_t.wait()


_packer = functools.partial(
    pl.kernel,
    out_type=(
        jax.ShapeDtypeStruct((BATCH, SEQ_LEN), jnp.int32),
        jax.ShapeDtypeStruct((BATCH, SEQ_LEN), jnp.int32),
        jax.ShapeDtypeStruct((BATCH, SEQ_LEN), jnp.int32),
    ),
    mesh=plsc.VectorSubcoreMesh(
        core_axis_name="c", subcore_axis_name="s",
        num_cores=2, num_subcores=16),
    scratch_types=[
        pltpu.VMEM((SEQ_LEN,), jnp.int32),
        pltpu.VMEM((SEQ_LEN,), jnp.int32),
        pltpu.VMEM((BATCH,), jnp.int32),
        pltpu.VMEM((BATCH,), jnp.int32),
        pltpu.VMEM((HALF,), jnp.int32),
        pltpu.VMEM((HALF,), jnp.int32),
        pltpu.VMEM((HALF,), jnp.int32),
        pltpu.SemaphoreType.DMA,
    ],
    compiler_params=pltpu.CompilerParams(needs_layout_passes=False),
)(_packer_body)


def kernel(tokens_a, len_a, tokens_b, len_b):
    return _packer(tokens_a, len_a, tokens_b, len_b)
